# baseline (speedup 1.0000x reference)
import functools

import jax
import jax.numpy as jnp
from jax import lax
from jax.experimental import pallas as pl
from jax.experimental.pallas import tpu as pltpu
from jax.experimental.pallas import tpu_sc as plsc

NC = 2
NS = 16
L = 16

N = 10000
D = 128
E = 320000

NPAD = 10240
ROWS_PER_TILE = NPAD // (NC * NS)
ROWS_PER_SC_TILE = NPAD // NS
EPAD = 327680
ECHUNKS = EPAD // (NC * NS * 128)
PCHUNKS = EPAD // (NS * 128)
DUMMY = NPAD - 1

_mesh = plsc.VectorSubcoreMesh(core_axis_name="c", subcore_axis_name="s")

_f32 = jnp.float32
_i32 = jnp.int32


def _rsqrt16(x):
    i = lax.bitcast_convert_type(x, _i32)
    i = jnp.int32(0x5F3759DF) - lax.shift_right_logical(i, 1)
    y = lax.bitcast_convert_type(i, _f32)
    for _ in range(3):
        y = y * (jnp.float32(1.5) - jnp.float32(0.5) * x * y * y)
    return y


def _bcast16(ref, idx_scalar):
    idx = jnp.zeros((L,), _i32) + idx_scalar
    return plsc.load_gather(ref, [idx])


def _prep_body(col_ref, cnt_ref, cnt_sh, colbuf, onesbuf, cntbuf, sem):
    cid = lax.axis_index("c")
    sid = lax.axis_index("s")
    wid = cid * NS + sid

    for i in range(ROWS_PER_SC_TILE // L):
        cntbuf[pl.ds(i * L, L)] = jnp.zeros((L,), _f32)
    for i in range(128 // L):
        onesbuf[pl.ds(i * L, L)] = jnp.ones((L,), _f32)
    pltpu.sync_copy(cntbuf, cnt_sh.at[pl.ds(sid * ROWS_PER_SC_TILE,
                                            ROWS_PER_SC_TILE)])
    pltpu.sync_copy(col_ref.at[pl.ds(wid * ECHUNKS, ECHUNKS)], colbuf)
    plsc.subcore_barrier()

    def count_step(j, carry):
        pltpu.sync_copy(onesbuf, cnt_sh.at[colbuf.at[j]], add=True)
        return carry
    lax.fori_loop(0, ECHUNKS, count_step, 0)
    plsc.subcore_barrier()

    base = sid * ROWS_PER_SC_TILE
    pltpu.sync_copy(cnt_sh.at[pl.ds(base, ROWS_PER_SC_TILE)],
                    cnt_ref.at[cid, pl.ds(base, ROWS_PER_SC_TILE)])


_prep = pl.kernel(
    _prep_body,
    out_type=jax.ShapeDtypeStruct((NC, NPAD), _f32),
    mesh=_mesh,
    compiler_params=pltpu.CompilerParams(needs_layout_passes=False),
    scratch_types=[
        pltpu.VMEM_SHARED((NPAD,), _f32),
        pltpu.VMEM((ECHUNKS, 128), _i32),
        pltpu.VMEM((128,), _f32),
        pltpu.VMEM((ROWS_PER_SC_TILE,), _f32),
        pltpu.SemaphoreType.DMA,
    ],
)


_NB = 2


def _scatter_body(y_ref, packed_ref, zeros_ref, p_ref,
                  acc, packedbuf, rc0, rc1, cc0, cc1, gb0, gb1,
                  g0, g1, ss0, ss1):
    cid = lax.axis_index("c")
    sid = lax.axis_index("s")
    wid = cid * NS + sid
    bufs = (gb0, gb1)
    rcs = (rc0, rc1)
    ccs = (cc0, cc1)
    gsems = (g0, g1)
    ssems = (ss0, ss1)

    nbase = sid * ROWS_PER_SC_TILE
    pltpu.sync_copy(zeros_ref.at[pl.ds(nbase, ROWS_PER_SC_TILE)],
                    acc.at[pl.ds(nbase, ROWS_PER_SC_TILE)])
    pltpu.sync_copy(packed_ref.at[pl.ds(wid * ECHUNKS * 128, ECHUNKS * 128)],
                    packedbuf)
    plsc.subcore_barrier()

    def unpack(i, rb, cb):
        for l in range(128 // L):
            pk = packedbuf[pl.ds(i * 128 + l * L, L)]
            rb[pl.ds(l * L, L)] = pk & jnp.int32(0xFFFF)
            cb[pl.ds(l * L, L)] = lax.shift_right_logical(pk, 16)

    for b in range(_NB):
        unpack(b, rcs[b], ccs[b])
        pltpu.async_copy(y_ref.at[rcs[b]], bufs[b], gsems[b])

    def block(k, carry):
        for b in range(_NB):
            pltpu.make_async_copy(y_ref.at[rcs[b]], bufs[b], gsems[b]).wait()
            pltpu.async_copy(bufs[b], acc.at[ccs[b]], ssems[b], add=True)
        for b in range(_NB):
            i = k * _NB + b
            nxt = i + _NB
            pltpu.make_async_copy(bufs[b], acc.at[ccs[b]], ssems[b]).wait()

            @pl.when(nxt < ECHUNKS)
            def _():
                unpack(nxt, rcs[b], ccs[b])
                pltpu.async_copy(y_ref.at[rcs[b]], bufs[b], gsems[b])
        return carry
    lax.fori_loop(0, ECHUNKS // _NB, block, 0)
    plsc.subcore_barrier()

    pltpu.sync_copy(acc.at[pl.ds(nbase, ROWS_PER_SC_TILE)],
                    p_ref.at[cid, pl.ds(nbase, ROWS_PER_SC_TILE)])


_scatter = pl.kernel(
    _scatter_body,
    out_type=jax.ShapeDtypeStruct((NC, NPAD, D), _f32),
    mesh=_mesh,
    compiler_params=pltpu.CompilerParams(needs_layout_passes=False),
    scratch_types=[
        pltpu.VMEM_SHARED((NPAD, D), _f32),
        pltpu.VMEM((ECHUNKS * 128,), _i32),
        pltpu.VMEM((128,), _i32),
        pltpu.VMEM((128,), _i32),
        pltpu.VMEM((128,), _i32),
        pltpu.VMEM((128,), _i32),
        pltpu.VMEM((128, D), _f32),
        pltpu.VMEM((128, D), _f32),
        pltpu.SemaphoreType.DMA,
        pltpu.SemaphoreType.DMA,
        pltpu.SemaphoreType.DMA,
        pltpu.SemaphoreType.DMA,
    ],
)


_TCR = 1024


def _prep_tc_body(cnt_ref, feat_ref, dis_ref, dis2_ref, y0_ref):
    deg = cnt_ref[0] + cnt_ref[1] + 1.0
    dis = lax.rsqrt(deg)
    dis_ref[...] = dis
    dis2_ref[...] = 1.0 / deg
    y0_ref[...] = dis * feat_ref[...]


_prep_tc = pl.pallas_call(
    _prep_tc_body,
    grid=(NPAD // _TCR,),
    in_specs=[
        pl.BlockSpec((2, _TCR, 1), lambda i: (0, i, 0)),
        pl.BlockSpec((_TCR, D), lambda i: (i, 0)),
    ],
    out_specs=[
        pl.BlockSpec((_TCR, 1), lambda i: (i, 0)),
        pl.BlockSpec((_TCR, 1), lambda i: (i, 0)),
        pl.BlockSpec((_TCR, D), lambda i: (i, 0)),
    ],
    out_shape=[
        jax.ShapeDtypeStruct((NPAD, 1), _f32),
        jax.ShapeDtypeStruct((NPAD, 1), _f32),
        jax.ShapeDtypeStruct((NPAD, D), _f32),
    ],
)


def _combine_tc_body(p_ref, y_ref, dis_ref, dis2_ref, x_ref, yn_ref):
    s = p_ref[0] + p_ref[1] + y_ref[...]
    x_ref[...] = dis_ref[...] * s
    yn_ref[...] = dis2_ref[...] * s


_combine_tc = pl.pallas_call(
    _combine_tc_body,
    grid=(NPAD // _TCR,),
    in_specs=[
        pl.BlockSpec((2, _TCR, D), lambda i: (0, i, 0)),
        pl.BlockSpec((_TCR, D), lambda i: (i, 0)),
        pl.BlockSpec((_TCR, 1), lambda i: (i, 0)),
        pl.BlockSpec((_TCR, 1), lambda i: (i, 0)),
    ],
    out_specs=[
        pl.BlockSpec((_TCR, D), lambda i: (i, 0)),
        pl.BlockSpec((_TCR, D), lambda i: (i, 0)),
    ],
    out_shape=[
        jax.ShapeDtypeStruct((NPAD, D), _f32),
        jax.ShapeDtypeStruct((NPAD, D), _f32),
    ],
)


K_HOPS = 8


def kernel(feature, edge_index):
    row = edge_index[0]
    col = edge_index[1]
    pad = EPAD - E
    padv = N + jnp.arange(pad, dtype=_i32) % (NPAD - N)
    rowf = jnp.concatenate([row, padv])
    colf = jnp.concatenate([col, padv])
    colp = colf.reshape(EPAD // 128, 128)
    packed = rowf | (colf << 16)
    featp = jnp.pad(feature, ((0, NPAD - N), (0, 0)))

    cnt = _prep(colp)
    dis, dis2, y2d = _prep_tc(cnt.reshape(NC, NPAD, 1), featp)

    zeros = jnp.zeros((NPAD, D), _f32)
    outs = [feature]
    for _ in range(K_HOPS):
        p = _scatter(y2d, packed, zeros)
        x, y2d = _combine_tc(p, y2d, dis, dis2)
        outs.append(x[:N])
    return jnp.concatenate(outs, axis=1)


# speedup vs baseline: 1.2906x; 1.2906x over previous
import functools

import jax
import jax.numpy as jnp
from jax import lax
from jax.experimental import pallas as pl
from jax.experimental.pallas import tpu as pltpu
from jax.experimental.pallas import tpu_sc as plsc

NC = 2
NS = 16
L = 16

N = 10000
D = 128
E = 320000

NPAD = 10240
ROWS_PER_TILE = NPAD // (NC * NS)
ROWS_PER_SC_TILE = NPAD // NS
EPAD = 327680
ECHUNKS = EPAD // (NC * NS * 128)
PCHUNKS = EPAD // (NS * 128)
DUMMY = NPAD - 1

_mesh = plsc.VectorSubcoreMesh(core_axis_name="c", subcore_axis_name="s")

_f32 = jnp.float32
_i32 = jnp.int32


def _rsqrt16(x):
    i = lax.bitcast_convert_type(x, _i32)
    i = jnp.int32(0x5F3759DF) - lax.shift_right_logical(i, 1)
    y = lax.bitcast_convert_type(i, _f32)
    for _ in range(3):
        y = y * (jnp.float32(1.5) - jnp.float32(0.5) * x * y * y)
    return y


def _bcast16(ref, idx_scalar):
    idx = jnp.zeros((L,), _i32) + idx_scalar
    return plsc.load_gather(ref, [idx])


def _prep_body(col_ref, cnt_ref, cnt_sh, colbuf, onesbuf, cntbuf, sem):
    cid = lax.axis_index("c")
    sid = lax.axis_index("s")
    wid = cid * NS + sid

    for i in range(ROWS_PER_SC_TILE // L):
        cntbuf[pl.ds(i * L, L)] = jnp.zeros((L,), _f32)
    for i in range(128 // L):
        onesbuf[pl.ds(i * L, L)] = jnp.ones((L,), _f32)
    pltpu.sync_copy(cntbuf, cnt_sh.at[pl.ds(sid * ROWS_PER_SC_TILE,
                                            ROWS_PER_SC_TILE)])
    pltpu.sync_copy(col_ref.at[pl.ds(wid * ECHUNKS, ECHUNKS)], colbuf)
    plsc.subcore_barrier()

    def count_step(j, carry):
        pltpu.sync_copy(onesbuf, cnt_sh.at[colbuf.at[j]], add=True)
        return carry
    lax.fori_loop(0, ECHUNKS, count_step, 0)
    plsc.subcore_barrier()

    base = sid * ROWS_PER_SC_TILE
    pltpu.sync_copy(cnt_sh.at[pl.ds(base, ROWS_PER_SC_TILE)],
                    cnt_ref.at[cid, pl.ds(base, ROWS_PER_SC_TILE)])


_prep = pl.kernel(
    _prep_body,
    out_type=jax.ShapeDtypeStruct((NC, NPAD), _f32),
    mesh=_mesh,
    compiler_params=pltpu.CompilerParams(needs_layout_passes=False),
    scratch_types=[
        pltpu.VMEM_SHARED((NPAD,), _f32),
        pltpu.VMEM((ECHUNKS, 128), _i32),
        pltpu.VMEM((128,), _f32),
        pltpu.VMEM((ROWS_PER_SC_TILE,), _f32),
        pltpu.SemaphoreType.DMA,
    ],
)


_NB = 2


def _scatter_body(y_ref, packed_ref, zeros_ref, p_ref,
                  acc, packedbuf, rc0, rc1, cc0, cc1, gb0, gb1,
                  g0, g1, ss0, ss1):
    cid = lax.axis_index("c")
    sid = lax.axis_index("s")
    wid = cid * NS + sid
    bufs = (gb0, gb1)
    rcs = (rc0, rc1)
    ccs = (cc0, cc1)
    gsems = (g0, g1)
    ssems = (ss0, ss1)

    nbase = sid * ROWS_PER_SC_TILE
    pltpu.sync_copy(zeros_ref.at[pl.ds(nbase, ROWS_PER_SC_TILE)],
                    acc.at[pl.ds(nbase, ROWS_PER_SC_TILE)])
    pltpu.sync_copy(packed_ref.at[pl.ds(wid * ECHUNKS * 128, ECHUNKS * 128)],
                    packedbuf)
    plsc.subcore_barrier()

    def unpack(i, rb, cb):
        for l in range(128 // L):
            pk = packedbuf[pl.ds(i * 128 + l * L, L)]
            rb[pl.ds(l * L, L)] = pk & jnp.int32(0xFFFF)
            cb[pl.ds(l * L, L)] = lax.shift_right_logical(pk, 16)

    for b in range(_NB):
        unpack(b, rcs[b], ccs[b])
        pltpu.async_copy(y_ref.at[rcs[b]], bufs[b], gsems[b])

    def block(k, carry):
        for b in range(_NB):
            i = k * _NB + b
            pltpu.make_async_copy(y_ref.at[rcs[b]], bufs[b], gsems[b]).wait()
            pltpu.sync_copy(bufs[b], acc.at[ccs[b]], add=True)
            nxt = i + _NB

            @pl.when(nxt < ECHUNKS)
            def _():
                unpack(nxt, rcs[b], ccs[b])
                pltpu.async_copy(y_ref.at[rcs[b]], bufs[b], gsems[b])
        return carry
    lax.fori_loop(0, ECHUNKS // _NB, block, 0)
    plsc.subcore_barrier()

    pltpu.sync_copy(acc.at[pl.ds(nbase, ROWS_PER_SC_TILE)],
                    p_ref.at[cid, pl.ds(nbase, ROWS_PER_SC_TILE)])


_scatter = pl.kernel(
    _scatter_body,
    out_type=jax.ShapeDtypeStruct((NC, NPAD, D), _f32),
    mesh=_mesh,
    compiler_params=pltpu.CompilerParams(needs_layout_passes=False),
    scratch_types=[
        pltpu.VMEM_SHARED((NPAD, D), _f32),
        pltpu.VMEM((ECHUNKS * 128,), _i32),
        pltpu.VMEM((128,), _i32),
        pltpu.VMEM((128,), _i32),
        pltpu.VMEM((128,), _i32),
        pltpu.VMEM((128,), _i32),
        pltpu.VMEM((128, D), _f32),
        pltpu.VMEM((128, D), _f32),
        pltpu.SemaphoreType.DMA,
        pltpu.SemaphoreType.DMA,
        pltpu.SemaphoreType.DMA,
        pltpu.SemaphoreType.DMA,
    ],
)


_TCR = 1024


def _prep_tc_body(cnt_ref, feat_ref, dis_ref, dis2_ref, y0_ref):
    deg = cnt_ref[0] + cnt_ref[1] + 1.0
    dis = lax.rsqrt(deg)
    dis_ref[...] = dis
    dis2_ref[...] = 1.0 / deg
    y0_ref[...] = dis * feat_ref[...]


_prep_tc = pl.pallas_call(
    _prep_tc_body,
    grid=(NPAD // _TCR,),
    in_specs=[
        pl.BlockSpec((2, _TCR, 1), lambda i: (0, i, 0)),
        pl.BlockSpec((_TCR, D), lambda i: (i, 0)),
    ],
    out_specs=[
        pl.BlockSpec((_TCR, 1), lambda i: (i, 0)),
        pl.BlockSpec((_TCR, 1), lambda i: (i, 0)),
        pl.BlockSpec((_TCR, D), lambda i: (i, 0)),
    ],
    out_shape=[
        jax.ShapeDtypeStruct((NPAD, 1), _f32),
        jax.ShapeDtypeStruct((NPAD, 1), _f32),
        jax.ShapeDtypeStruct((NPAD, D), _f32),
    ],
)


def _combine_tc_body(p_ref, y_ref, dis_ref, dis2_ref, x_ref, yn_ref):
    s = p_ref[0] + p_ref[1] + y_ref[...]
    x_ref[...] = dis_ref[...] * s
    yn_ref[...] = dis2_ref[...] * s


_combine_tc = pl.pallas_call(
    _combine_tc_body,
    grid=(NPAD // _TCR,),
    in_specs=[
        pl.BlockSpec((2, _TCR, D), lambda i: (0, i, 0)),
        pl.BlockSpec((_TCR, D), lambda i: (i, 0)),
        pl.BlockSpec((_TCR, 1), lambda i: (i, 0)),
        pl.BlockSpec((_TCR, 1), lambda i: (i, 0)),
    ],
    out_specs=[
        pl.BlockSpec((_TCR, D), lambda i: (i, 0)),
        pl.BlockSpec((_TCR, D), lambda i: (i, 0)),
    ],
    out_shape=[
        jax.ShapeDtypeStruct((NPAD, D), _f32),
        jax.ShapeDtypeStruct((NPAD, D), _f32),
    ],
)


K_HOPS = 8


def kernel(feature, edge_index):
    row = edge_index[0]
    col = edge_index[1]
    pad = EPAD - E
    padv = N + jnp.arange(pad, dtype=_i32) % (NPAD - N)
    rowf = jnp.concatenate([row, padv])
    colf = jnp.concatenate([col, padv])
    colp = colf.reshape(EPAD // 128, 128)
    packed = rowf | (colf << 16)
    featp = jnp.pad(feature, ((0, NPAD - N), (0, 0)))

    cnt = _prep(colp)
    dis, dis2, y2d = _prep_tc(cnt.reshape(NC, NPAD, 1), featp)

    zeros = jnp.zeros((NPAD, D), _f32)
    outs = [feature]
    for _ in range(K_HOPS):
        p = _scatter(y2d, packed, zeros)
        x, y2d = _combine_tc(p, y2d, dis, dis2)
        outs.append(x[:N])
    return jnp.concatenate(outs, axis=1)


# speedup vs baseline: 1.2989x; 1.0064x over previous
import functools

import jax
import jax.numpy as jnp
from jax import lax
from jax.experimental import pallas as pl
from jax.experimental.pallas import tpu as pltpu
from jax.experimental.pallas import tpu_sc as plsc

NC = 2
NS = 16
L = 16

N = 10000
D = 128
E = 320000

NPAD = 10240
ROWS_PER_TILE = NPAD // (NC * NS)
ROWS_PER_SC_TILE = NPAD // NS
EPAD = 327680
ECHUNKS = EPAD // (NC * NS * 128)
PCHUNKS = EPAD // (NS * 128)
DUMMY = NPAD - 1

_mesh = plsc.VectorSubcoreMesh(core_axis_name="c", subcore_axis_name="s")

_f32 = jnp.float32
_i32 = jnp.int32


def _rsqrt16(x):
    i = lax.bitcast_convert_type(x, _i32)
    i = jnp.int32(0x5F3759DF) - lax.shift_right_logical(i, 1)
    y = lax.bitcast_convert_type(i, _f32)
    for _ in range(3):
        y = y * (jnp.float32(1.5) - jnp.float32(0.5) * x * y * y)
    return y


def _bcast16(ref, idx_scalar):
    idx = jnp.zeros((L,), _i32) + idx_scalar
    return plsc.load_gather(ref, [idx])


def _prep_body(col_ref, cnt_ref, cnt_sh, colbuf, onesbuf, cntbuf, sem):
    cid = lax.axis_index("c")
    sid = lax.axis_index("s")
    wid = cid * NS + sid

    for i in range(ROWS_PER_SC_TILE // L):
        cntbuf[pl.ds(i * L, L)] = jnp.zeros((L,), _f32)
    for i in range(128 // L):
        onesbuf[pl.ds(i * L, L)] = jnp.ones((L,), _f32)
    pltpu.sync_copy(cntbuf, cnt_sh.at[pl.ds(sid * ROWS_PER_SC_TILE,
                                            ROWS_PER_SC_TILE)])
    pltpu.sync_copy(col_ref.at[pl.ds(wid * ECHUNKS, ECHUNKS)], colbuf)
    plsc.subcore_barrier()

    def count_step(j, carry):
        pltpu.sync_copy(onesbuf, cnt_sh.at[colbuf.at[j]], add=True)
        return carry
    lax.fori_loop(0, ECHUNKS, count_step, 0)
    plsc.subcore_barrier()

    base = sid * ROWS_PER_SC_TILE
    pltpu.sync_copy(cnt_sh.at[pl.ds(base, ROWS_PER_SC_TILE)],
                    cnt_ref.at[cid, pl.ds(base, ROWS_PER_SC_TILE)])


_prep = pl.kernel(
    _prep_body,
    out_type=jax.ShapeDtypeStruct((NC, NPAD), _f32),
    mesh=_mesh,
    compiler_params=pltpu.CompilerParams(needs_layout_passes=False),
    scratch_types=[
        pltpu.VMEM_SHARED((NPAD,), _f32),
        pltpu.VMEM((ECHUNKS, 128), _i32),
        pltpu.VMEM((128,), _f32),
        pltpu.VMEM((ROWS_PER_SC_TILE,), _f32),
        pltpu.SemaphoreType.DMA,
    ],
)


_NB = 2


def _scatter_body(y_ref, packed_ref, zeros_ref, p_ref,
                  acc, packedbuf, rc0, rc1, rc2, rc3, cc0, cc1, cc2, cc3,
                  gb0, gb1, g0, g1):
    cid = lax.axis_index("c")
    sid = lax.axis_index("s")
    wid = cid * NS + sid
    bufs = (gb0, gb1)
    rcs = ((rc0, rc1), (rc2, rc3))
    ccs = ((cc0, cc1), (cc2, cc3))
    gsems = (g0, g1)

    nbase = sid * ROWS_PER_SC_TILE
    @pl.when(cid == 0)
    def _():
        pltpu.sync_copy(y_ref.at[pl.ds(nbase, ROWS_PER_SC_TILE)],
                        acc.at[pl.ds(nbase, ROWS_PER_SC_TILE)])

    @pl.when(cid == 1)
    def _():
        pltpu.sync_copy(zeros_ref.at[pl.ds(nbase, ROWS_PER_SC_TILE)],
                        acc.at[pl.ds(nbase, ROWS_PER_SC_TILE)])

    pltpu.sync_copy(packed_ref.at[pl.ds(wid * ECHUNKS * 128, ECHUNKS * 128)],
                    packedbuf)
    plsc.subcore_barrier()

    def unpack(i, rb, cb):
        for l in range(128 // L):
            pk = packedbuf[pl.ds(i * 128 + l * L, L)]
            rb[pl.ds(l * L, L)] = pk & jnp.int32(0xFFFF)
            cb[pl.ds(l * L, L)] = lax.shift_right_logical(pk, 16)

    for b in range(_NB):
        unpack(b, rcs[0][b], ccs[0][b])
        pltpu.async_copy(y_ref.at[rcs[0][b]], bufs[b], gsems[b])

    def block(k, carry):
        par = lax.rem(k, 2)
        nxtpar = lax.rem(k + 1, 2)
        for b in range(_NB):
            nxt = (k + 1) * _NB + b

            @pl.when(nxt < ECHUNKS)
            def _():
                @pl.when(nxtpar == 0)
                def _():
                    unpack(nxt, rcs[0][b], ccs[0][b])

                @pl.when(nxtpar == 1)
                def _():
                    unpack(nxt, rcs[1][b], ccs[1][b])
        for par_v in range(2):
            @pl.when(par == par_v)
            def _():
                for b in range(_NB):
                    nxt = (k + 1) * _NB + b
                    pltpu.make_async_copy(y_ref.at[rcs[par_v][b]], bufs[b],
                                          gsems[b]).wait()
                    pltpu.sync_copy(bufs[b], acc.at[ccs[par_v][b]], add=True)

                    @pl.when(nxt < ECHUNKS)
                    def _():
                        pltpu.async_copy(y_ref.at[rcs[1 - par_v][b]], bufs[b],
                                         gsems[b])
        return carry
    lax.fori_loop(0, ECHUNKS // _NB, block, 0)
    plsc.subcore_barrier()

    pltpu.sync_copy(acc.at[pl.ds(nbase, ROWS_PER_SC_TILE)],
                    p_ref.at[cid, pl.ds(nbase, ROWS_PER_SC_TILE)])


_scatter = pl.kernel(
    _scatter_body,
    out_type=jax.ShapeDtypeStruct((NC, NPAD, D), _f32),
    mesh=_mesh,
    compiler_params=pltpu.CompilerParams(needs_layout_passes=False),
    scratch_types=[
        pltpu.VMEM_SHARED((NPAD, D), _f32),
        pltpu.VMEM((ECHUNKS * 128,), _i32),
        pltpu.VMEM((128,), _i32),
        pltpu.VMEM((128,), _i32),
        pltpu.VMEM((128,), _i32),
        pltpu.VMEM((128,), _i32),
        pltpu.VMEM((128,), _i32),
        pltpu.VMEM((128,), _i32),
        pltpu.VMEM((128,), _i32),
        pltpu.VMEM((128,), _i32),
        pltpu.VMEM((128, D), _f32),
        pltpu.VMEM((128, D), _f32),
        pltpu.SemaphoreType.DMA,
        pltpu.SemaphoreType.DMA,
    ],
)


_TCR = 1024


def _prep_tc_body(cnt_ref, feat_ref, dis_ref, dis2_ref, y0_ref):
    deg = cnt_ref[0] + cnt_ref[1] + 1.0
    dis = lax.rsqrt(deg)
    dis_ref[...] = dis
    dis2_ref[...] = 1.0 / deg
    y0_ref[...] = dis * feat_ref[...]


_prep_tc = pl.pallas_call(
    _prep_tc_body,
    grid=(NPAD // _TCR,),
    in_specs=[
        pl.BlockSpec((2, _TCR, 1), lambda i: (0, i, 0)),
        pl.BlockSpec((_TCR, D), lambda i: (i, 0)),
    ],
    out_specs=[
        pl.BlockSpec((_TCR, 1), lambda i: (i, 0)),
        pl.BlockSpec((_TCR, 1), lambda i: (i, 0)),
        pl.BlockSpec((_TCR, D), lambda i: (i, 0)),
    ],
    out_shape=[
        jax.ShapeDtypeStruct((NPAD, 1), _f32),
        jax.ShapeDtypeStruct((NPAD, 1), _f32),
        jax.ShapeDtypeStruct((NPAD, D), _f32),
    ],
)


def _combine_tc_body(p_ref, dis_ref, dis2_ref, x_ref, yn_ref):
    s = p_ref[0] + p_ref[1]
    x_ref[...] = dis_ref[...] * s
    yn_ref[...] = dis2_ref[...] * s


_combine_tc = pl.pallas_call(
    _combine_tc_body,
    grid=(NPAD // _TCR,),
    in_specs=[
        pl.BlockSpec((2, _TCR, D), lambda i: (0, i, 0)),
        pl.BlockSpec((_TCR, 1), lambda i: (i, 0)),
        pl.BlockSpec((_TCR, 1), lambda i: (i, 0)),
    ],
    out_specs=[
        pl.BlockSpec((_TCR, D), lambda i: (i, 0)),
        pl.BlockSpec((_TCR, D), lambda i: (i, 0)),
    ],
    out_shape=[
        jax.ShapeDtypeStruct((NPAD, D), _f32),
        jax.ShapeDtypeStruct((NPAD, D), _f32),
    ],
)


K_HOPS = 8


def kernel(feature, edge_index):
    row = edge_index[0]
    col = edge_index[1]
    pad = EPAD - E
    padv = N + jnp.arange(pad, dtype=_i32) % (NPAD - N)
    rowf = jnp.concatenate([row, padv])
    colf = jnp.concatenate([col, padv])
    colp = colf.reshape(EPAD // 128, 128)
    packed = rowf | (colf << 16)
    featp = jnp.pad(feature, ((0, NPAD - N), (0, 0)))

    cnt = _prep(colp)
    dis, dis2, y2d = _prep_tc(cnt.reshape(NC, NPAD, 1), featp)

    zeros = jnp.zeros((NPAD, D), _f32)
    outs = [feature]
    for _ in range(K_HOPS):
        p = _scatter(y2d, packed, zeros)
        x, y2d = _combine_tc(p, dis, dis2)
        outs.append(x[:N])
    return jnp.concatenate(outs, axis=1)


# speedup vs baseline: 1.3132x; 1.0110x over previous
import jax
import jax.numpy as jnp
from jax import lax
from jax.experimental import pallas as pl
from jax.experimental.pallas import tpu as pltpu
from jax.experimental.pallas import tpu_sc as plsc

NC = 2
NS = 16
L = 16

N = 10000
D = 128
E = 320000

NPAD = 10240
ROWS_PER_SC_TILE = NPAD // NS
EPAD = 327680
ECHUNKS = EPAD // (NC * NS * 128)

_mesh = plsc.VectorSubcoreMesh(core_axis_name="c", subcore_axis_name="s")

_f32 = jnp.float32
_i32 = jnp.int32


def _prep_body(col_ref, cnt_ref, cnt_sh, colbuf, onesbuf, cntbuf, sem):
    cid = lax.axis_index("c")
    sid = lax.axis_index("s")
    wid = cid * NS + sid

    for i in range(ROWS_PER_SC_TILE // L):
        cntbuf[pl.ds(i * L, L)] = jnp.zeros((L,), _f32)
    for i in range(128 // L):
        onesbuf[pl.ds(i * L, L)] = jnp.ones((L,), _f32)
    pltpu.sync_copy(cntbuf, cnt_sh.at[pl.ds(sid * ROWS_PER_SC_TILE,
                                            ROWS_PER_SC_TILE)])
    pltpu.sync_copy(col_ref.at[pl.ds(wid * ECHUNKS, ECHUNKS)], colbuf)
    plsc.subcore_barrier()

    def count_step(j, carry):
        pltpu.sync_copy(onesbuf, cnt_sh.at[colbuf.at[j]], add=True)
        return carry
    lax.fori_loop(0, ECHUNKS, count_step, 0)
    plsc.subcore_barrier()

    base = sid * ROWS_PER_SC_TILE
    pltpu.sync_copy(cnt_sh.at[pl.ds(base, ROWS_PER_SC_TILE)],
                    cnt_ref.at[cid, pl.ds(base, ROWS_PER_SC_TILE)])


_prep = pl.kernel(
    _prep_body,
    out_type=jax.ShapeDtypeStruct((NC, NPAD), _f32),
    mesh=_mesh,
    compiler_params=pltpu.CompilerParams(needs_layout_passes=False),
    scratch_types=[
        pltpu.VMEM_SHARED((NPAD,), _f32),
        pltpu.VMEM((ECHUNKS, 128), _i32),
        pltpu.VMEM((128,), _f32),
        pltpu.VMEM((ROWS_PER_SC_TILE,), _f32),
        pltpu.SemaphoreType.DMA,
    ],
)


_NB = 2


def _scatter_body(y_ref, packed_ref, p_ref,
                  acc, packedbuf, rc0, rc1, rc2, rc3, cc0, cc1, cc2, cc3,
                  gb0, gb1, g0, g1):
    cid = lax.axis_index("c")
    sid = lax.axis_index("s")
    wid = cid * NS + sid
    bufs = (gb0, gb1)
    rcs = ((rc0, rc1), (rc2, rc3))
    ccs = ((cc0, cc1), (cc2, cc3))
    gsems = (g0, g1)

    nbase = sid * ROWS_PER_SC_TILE
    @pl.when(cid == 0)
    def _():
        pltpu.sync_copy(y_ref.at[pl.ds(nbase, ROWS_PER_SC_TILE)],
                        acc.at[pl.ds(nbase, ROWS_PER_SC_TILE)])

    @pl.when(cid == 1)
    def _():
        def zrow(r, carry):
            for l in range(D // L):
                gb0[r, pl.ds(l * L, L)] = jnp.zeros((L,), _f32)
            return carry
        lax.fori_loop(0, 128, zrow, 0)
        for t in range(ROWS_PER_SC_TILE // 128):
            pltpu.sync_copy(gb0, acc.at[pl.ds(nbase + t * 128, 128)])

    pltpu.sync_copy(packed_ref.at[pl.ds(wid * ECHUNKS * 128, ECHUNKS * 128)],
                    packedbuf)
    plsc.subcore_barrier()

    def unpack(i, rb, cb):
        for l in range(128 // L):
            pk = packedbuf[pl.ds(i * 128 + l * L, L)]
            rb[pl.ds(l * L, L)] = pk & jnp.int32(0xFFFF)
            cb[pl.ds(l * L, L)] = lax.shift_right_logical(pk, 16)

    for b in range(_NB):
        unpack(b, rcs[0][b], ccs[0][b])
        pltpu.async_copy(y_ref.at[rcs[0][b]], bufs[b], gsems[b])

    def block(k, carry):
        par = lax.rem(k, 2)
        nxtpar = lax.rem(k + 1, 2)
        for b in range(_NB):
            nxt = (k + 1) * _NB + b

            @pl.when(nxt < ECHUNKS)
            def _():
                @pl.when(nxtpar == 0)
                def _():
                    unpack(nxt, rcs[0][b], ccs[0][b])

                @pl.when(nxtpar == 1)
                def _():
                    unpack(nxt, rcs[1][b], ccs[1][b])
        for par_v in range(2):
            @pl.when(par == par_v)
            def _():
                for b in range(_NB):
                    nxt = (k + 1) * _NB + b
                    pltpu.make_async_copy(y_ref.at[rcs[par_v][b]], bufs[b],
                                          gsems[b]).wait()
                    pltpu.sync_copy(bufs[b], acc.at[ccs[par_v][b]], add=True)

                    @pl.when(nxt < ECHUNKS)
                    def _():
                        pltpu.async_copy(y_ref.at[rcs[1 - par_v][b]], bufs[b],
                                         gsems[b])
        return carry
    lax.fori_loop(0, ECHUNKS // _NB, block, 0)
    plsc.subcore_barrier()

    pltpu.sync_copy(acc.at[pl.ds(nbase, ROWS_PER_SC_TILE)],
                    p_ref.at[cid, pl.ds(nbase, ROWS_PER_SC_TILE)])


_scatter = pl.kernel(
    _scatter_body,
    out_type=jax.ShapeDtypeStruct((NC, NPAD, D), _f32),
    mesh=_mesh,
    compiler_params=pltpu.CompilerParams(needs_layout_passes=False),
    scratch_types=[
        pltpu.VMEM_SHARED((NPAD, D), _f32),
        pltpu.VMEM((ECHUNKS * 128,), _i32),
        pltpu.VMEM((128,), _i32),
        pltpu.VMEM((128,), _i32),
        pltpu.VMEM((128,), _i32),
        pltpu.VMEM((128,), _i32),
        pltpu.VMEM((128,), _i32),
        pltpu.VMEM((128,), _i32),
        pltpu.VMEM((128,), _i32),
        pltpu.VMEM((128,), _i32),
        pltpu.VMEM((128, D), _f32),
        pltpu.VMEM((128, D), _f32),
        pltpu.SemaphoreType.DMA,
        pltpu.SemaphoreType.DMA,
    ],
)


_TCR = 1024


def _prep_tc_body(cnt_ref, feat_ref, dis_ref, dis2_ref, y0_ref):
    deg = cnt_ref[0] + cnt_ref[1] + 1.0
    dis = lax.rsqrt(deg)
    dis_ref[...] = dis
    dis2_ref[...] = 1.0 / deg
    y0_ref[...] = dis * feat_ref[...]


_prep_tc = pl.pallas_call(
    _prep_tc_body,
    grid=(NPAD // _TCR,),
    in_specs=[
        pl.BlockSpec((2, _TCR, 1), lambda i: (0, i, 0)),
        pl.BlockSpec((_TCR, D), lambda i: (i, 0)),
    ],
    out_specs=[
        pl.BlockSpec((_TCR, 1), lambda i: (i, 0)),
        pl.BlockSpec((_TCR, 1), lambda i: (i, 0)),
        pl.BlockSpec((_TCR, D), lambda i: (i, 0)),
    ],
    out_shape=[
        jax.ShapeDtypeStruct((NPAD, 1), _f32),
        jax.ShapeDtypeStruct((NPAD, 1), _f32),
        jax.ShapeDtypeStruct((NPAD, D), _f32),
    ],
)


def _combine_tc_body(p_ref, dis_ref, dis2_ref, x_ref, yn_ref):
    s = p_ref[0] + p_ref[1]
    x_ref[...] = dis_ref[...] * s
    yn_ref[...] = dis2_ref[...] * s


_combine_tc = pl.pallas_call(
    _combine_tc_body,
    grid=(NPAD // _TCR,),
    in_specs=[
        pl.BlockSpec((2, _TCR, D), lambda i: (0, i, 0)),
        pl.BlockSpec((_TCR, 1), lambda i: (i, 0)),
        pl.BlockSpec((_TCR, 1), lambda i: (i, 0)),
    ],
    out_specs=[
        pl.BlockSpec((_TCR, D), lambda i: (i, 0)),
        pl.BlockSpec((_TCR, D), lambda i: (i, 0)),
    ],
    out_shape=[
        jax.ShapeDtypeStruct((NPAD, D), _f32),
        jax.ShapeDtypeStruct((NPAD, D), _f32),
    ],
)


K_HOPS = 8


def kernel(feature, edge_index):
    row = edge_index[0]
    col = edge_index[1]
    pad = EPAD - E
    padv = N + jnp.arange(pad, dtype=_i32) % (NPAD - N)
    rowf = jnp.concatenate([row, padv])
    colf = jnp.concatenate([col, padv])
    colp = colf.reshape(EPAD // 128, 128)
    packed = rowf | (colf << 16)
    featp = jnp.pad(feature, ((0, NPAD - N), (0, 0)))

    cnt = _prep(colp)
    dis, dis2, y2d = _prep_tc(cnt.reshape(NC, NPAD, 1), featp)

    outs = [feature]
    for _ in range(K_HOPS):
        p = _scatter(y2d, packed)
        x, y2d = _combine_tc(p, dis, dis2)
        outs.append(x[:N])
    return jnp.concatenate(outs, axis=1)


# speedup vs baseline: 1.3224x; 1.0070x over previous
import jax
import jax.numpy as jnp
from jax import lax
from jax.experimental import pallas as pl
from jax.experimental.pallas import tpu as pltpu
from jax.experimental.pallas import tpu_sc as plsc

NC = 2
NS = 16
L = 16

N = 10000
D = 128
E = 320000

NPAD = 10240
ROWS_PER_SC_TILE = NPAD // NS
EPAD = 327680
ECHUNKS = EPAD // (NC * NS * 128)

_mesh = plsc.VectorSubcoreMesh(core_axis_name="c", subcore_axis_name="s")

_f32 = jnp.float32
_i32 = jnp.int32


def _prep_body(col_ref, cnt_ref, cnt_sh, colbuf, onesbuf, cntbuf, sem):
    cid = lax.axis_index("c")
    sid = lax.axis_index("s")
    wid = cid * NS + sid

    for i in range(ROWS_PER_SC_TILE // L):
        cntbuf[pl.ds(i * L, L)] = jnp.zeros((L,), _f32)
    for i in range(128 // L):
        onesbuf[pl.ds(i * L, L)] = jnp.ones((L,), _f32)
    pltpu.sync_copy(cntbuf, cnt_sh.at[pl.ds(sid * ROWS_PER_SC_TILE,
                                            ROWS_PER_SC_TILE)])
    pltpu.sync_copy(col_ref.at[pl.ds(wid * ECHUNKS, ECHUNKS)], colbuf)
    plsc.subcore_barrier()

    def count_step(j, carry):
        pltpu.sync_copy(onesbuf, cnt_sh.at[colbuf.at[j]], add=True)
        return carry
    lax.fori_loop(0, ECHUNKS, count_step, 0)
    plsc.subcore_barrier()

    base = sid * ROWS_PER_SC_TILE
    pltpu.sync_copy(cnt_sh.at[pl.ds(base, ROWS_PER_SC_TILE)],
                    cnt_ref.at[cid, pl.ds(base, ROWS_PER_SC_TILE)])


_prep = pl.kernel(
    _prep_body,
    out_type=jax.ShapeDtypeStruct((NC, NPAD), _f32),
    mesh=_mesh,
    compiler_params=pltpu.CompilerParams(needs_layout_passes=False),
    scratch_types=[
        pltpu.VMEM_SHARED((NPAD,), _f32),
        pltpu.VMEM((ECHUNKS, 128), _i32),
        pltpu.VMEM((128,), _f32),
        pltpu.VMEM((ROWS_PER_SC_TILE,), _f32),
        pltpu.SemaphoreType.DMA,
    ],
)


_NB = 2


def _scatter_body(y_ref, packed_ref, p_ref,
                  acc, packedbuf, rc0, rc1, rc2, rc3, cc0, cc1, cc2, cc3,
                  gb0, gb1, g0, g1):
    cid = lax.axis_index("c")
    sid = lax.axis_index("s")
    wid = cid * NS + sid
    bufs = (gb0, gb1)
    rcs = ((rc0, rc1), (rc2, rc3))
    ccs = ((cc0, cc1), (cc2, cc3))
    gsems = (g0, g1)

    nbase = sid * ROWS_PER_SC_TILE
    @pl.when(cid == 0)
    def _():
        pltpu.sync_copy(y_ref.at[pl.ds(nbase, ROWS_PER_SC_TILE)],
                        acc.at[pl.ds(nbase, ROWS_PER_SC_TILE)])

    @pl.when(cid == 1)
    def _():
        def zrow(r, carry):
            for l in range(D // L):
                gb0[r, pl.ds(l * L, L)] = jnp.zeros((L,), _f32)
            return carry
        lax.fori_loop(0, 128, zrow, 0)
        for t in range(ROWS_PER_SC_TILE // 128):
            pltpu.sync_copy(gb0, acc.at[pl.ds(nbase + t * 128, 128)])

    pltpu.sync_copy(packed_ref.at[pl.ds(wid * ECHUNKS * 128, ECHUNKS * 128)],
                    packedbuf)
    plsc.subcore_barrier()

    def unpack(i, rb, cb):
        for l in range(128 // L):
            pk = packedbuf[pl.ds(i * 128 + l * L, L)]
            rb[pl.ds(l * L, L)] = pk & jnp.int32(0xFFFF)
            cb[pl.ds(l * L, L)] = lax.shift_right_logical(pk, 16)

    for b in range(_NB):
        unpack(b, rcs[0][b], ccs[0][b])
        pltpu.async_copy(y_ref.at[rcs[0][b]], bufs[b], gsems[b])

    def block(k, carry):
        par = lax.rem(k, 2)
        nxtpar = lax.rem(k + 1, 2)
        for b in range(_NB):
            nxt = (k + 1) * _NB + b

            @pl.when(nxt < ECHUNKS)
            def _():
                @pl.when(nxtpar == 0)
                def _():
                    unpack(nxt, rcs[0][b], ccs[0][b])

                @pl.when(nxtpar == 1)
                def _():
                    unpack(nxt, rcs[1][b], ccs[1][b])
        for par_v in range(2):
            @pl.when(par == par_v)
            def _():
                for b in range(_NB):
                    nxt = (k + 1) * _NB + b
                    pltpu.make_async_copy(y_ref.at[rcs[par_v][b]], bufs[b],
                                          gsems[b]).wait()
                    pltpu.sync_copy(bufs[b], acc.at[ccs[par_v][b]], add=True)

                    @pl.when(nxt < ECHUNKS)
                    def _():
                        pltpu.async_copy(y_ref.at[rcs[1 - par_v][b]], bufs[b],
                                         gsems[b])
        return carry
    lax.fori_loop(0, ECHUNKS // _NB, block, 0)
    plsc.subcore_barrier()

    pltpu.sync_copy(acc.at[pl.ds(nbase, ROWS_PER_SC_TILE)],
                    p_ref.at[cid, pl.ds(nbase, ROWS_PER_SC_TILE)])


_scatter = pl.kernel(
    _scatter_body,
    out_type=jax.ShapeDtypeStruct((NC, NPAD, D), _f32),
    mesh=_mesh,
    compiler_params=pltpu.CompilerParams(needs_layout_passes=False),
    scratch_types=[
        pltpu.VMEM_SHARED((NPAD, D), _f32),
        pltpu.VMEM((ECHUNKS * 128,), _i32),
        pltpu.VMEM((128,), _i32),
        pltpu.VMEM((128,), _i32),
        pltpu.VMEM((128,), _i32),
        pltpu.VMEM((128,), _i32),
        pltpu.VMEM((128,), _i32),
        pltpu.VMEM((128,), _i32),
        pltpu.VMEM((128,), _i32),
        pltpu.VMEM((128,), _i32),
        pltpu.VMEM((128, D), _f32),
        pltpu.VMEM((128, D), _f32),
        pltpu.SemaphoreType.DMA,
        pltpu.SemaphoreType.DMA,
    ],
)


_TCR = 1024


def _prep_tc_body(cnt_ref, feat_ref, dis_ref, dis2_ref, y0_ref):
    deg = cnt_ref[0] + cnt_ref[1] + 1.0
    dis = lax.rsqrt(deg)
    dis_ref[...] = dis
    dis2_ref[...] = 1.0 / deg
    y0_ref[...] = dis * feat_ref[...]


_prep_tc = pl.pallas_call(
    _prep_tc_body,
    grid=(NPAD // _TCR,),
    in_specs=[
        pl.BlockSpec((2, _TCR, 1), lambda i: (0, i, 0)),
        pl.BlockSpec((_TCR, D), lambda i: (i, 0)),
    ],
    out_specs=[
        pl.BlockSpec((_TCR, 1), lambda i: (i, 0)),
        pl.BlockSpec((_TCR, 1), lambda i: (i, 0)),
        pl.BlockSpec((_TCR, D), lambda i: (i, 0)),
    ],
    out_shape=[
        jax.ShapeDtypeStruct((NPAD, 1), _f32),
        jax.ShapeDtypeStruct((NPAD, 1), _f32),
        jax.ShapeDtypeStruct((NPAD, D), _f32),
    ],
)


def _combine_tc_body(big_ref, p_ref, dis_ref, dis2_ref, bigout_ref, yn_ref):
    del big_ref
    s = p_ref[0] + p_ref[1]
    bigout_ref[...] = dis_ref[...] * s
    yn_ref[...] = dis2_ref[...] * s


def _make_combine(hop):
    col = hop + 1
    return pl.pallas_call(
        _combine_tc_body,
        grid=(NPAD // _TCR,),
        in_specs=[
            pl.BlockSpec(memory_space=pl.ANY),
            pl.BlockSpec((2, _TCR, D), lambda i: (0, i, 0)),
            pl.BlockSpec((_TCR, 1), lambda i: (i, 0)),
            pl.BlockSpec((_TCR, 1), lambda i: (i, 0)),
        ],
        out_specs=[
            pl.BlockSpec((_TCR, D), lambda i, c=col: (i, c)),
            pl.BlockSpec((_TCR, D), lambda i: (i, 0)),
        ],
        out_shape=[
            jax.ShapeDtypeStruct((N, (K_HOPS + 1) * D), _f32),
            jax.ShapeDtypeStruct((NPAD, D), _f32),
        ],
        input_output_aliases={0: 0},
    )


K_HOPS = 8
_combines = [_make_combine(k) for k in range(K_HOPS)]


def kernel(feature, edge_index):
    row = edge_index[0]
    col = edge_index[1]
    pad = EPAD - E
    padv = N + jnp.arange(pad, dtype=_i32) % (NPAD - N)
    rowf = jnp.concatenate([row, padv])
    colf = jnp.concatenate([col, padv])
    colp = colf.reshape(EPAD // 128, 128)
    packed = rowf | (colf << 16)
    featp = jnp.pad(feature, ((0, NPAD - N), (0, 0)))

    cnt = _prep(colp)
    dis, dis2, y2d = _prep_tc(cnt.reshape(NC, NPAD, 1), featp)

    big = jnp.pad(feature, ((0, 0), (0, K_HOPS * D)))
    for k in range(K_HOPS):
        p = _scatter(y2d, packed)
        big, y2d = _combines[k](big, p, dis, dis2)
    return big
